# Initial kernel scaffold; baseline (speedup 1.0000x reference)
import jax
import jax.numpy as jnp
from jax.experimental import pallas as pl


def kernel(z, codebook):
    raise NotImplementedError("write your pallas kernel here")


# speedup vs baseline: 1.6963x; 1.6963x over previous
import numpy as np
import jax
import jax.numpy as jnp
from jax.experimental import pallas as pl
from jax.experimental.pallas import tpu as pltpu

_B, _D, _H, _W = 8, 64, 14, 14
_P = _H * _W
_N = _B * _P
_M = 1024

_consts = {}


def _loss_consts():
    if "loss" not in _consts:
        nb = _D + 1
        q = [(_P * i) // _D for i in range(nb)]
        r = [(_P * i) % _D for i in range(nb)]
        lmat = np.tril(np.ones((_D, _D), np.float32))
        fc = np.zeros((nb, _P), np.float32)
        qsel = np.zeros((_P, nb), np.float32)
        wc = np.zeros((_D, nb), np.float32)
        for i in range(nb):
            fc[i, : q[i]] = 1.0
            if r[i] > 0:
                qsel[q[i], i] = 1.0
                wc[r[i] - 1, i] = 1.0
        _consts["loss"] = (lmat, fc, qsel, wc)
    return _consts["loss"]


def _rvn_bdp():
    if "rvn" not in _consts:
        with jax.ensure_compile_time_eval():
            rv = jax.random.normal(
                jax.random.key(42), (_N, _D), dtype=jnp.float32)
            nrm = jnp.maximum(jnp.linalg.norm(rv, axis=-1, keepdims=True), 1e-9)
            rvn = np.asarray(rv / nrm)
        _consts["rvn"] = np.ascontiguousarray(
            np.transpose(rvn.reshape(_B, _P, _D), (0, 2, 1))
        )
    return _consts["rvn"]


def _nsvq_body(z_ref, cb_ref, rvn_ref, lmat_ref, fc_ref, qsel_ref, wc_ref,
               zq_ref, ind_ref, loss_ref):
    b = pl.program_id(0)
    zb = z_ref[0]
    cb = cb_ref[...]

    g = jax.lax.dot_general(
        cb, zb, (((1,), (0,)), ((), ())), preferred_element_type=jnp.float32
    )
    x2 = jnp.sum(zb * zb, axis=0, keepdims=True)
    y2 = jnp.sum(cb * cb, axis=1, keepdims=True)
    dist = (x2 + y2) - 2.0 * g

    dmin = jnp.min(dist, axis=0, keepdims=True)
    rows = jax.lax.broadcasted_iota(jnp.int32, (_M, _P), 0)
    ind = jnp.min(jnp.where(dist == dmin, rows, _M), axis=0)
    ind_ref[0, 0, :] = ind

    s = jnp.sqrt(jnp.maximum(dmin, 0.0))
    zq = zb + rvn_ref[0] * s
    zq_ref[0] = zq

    cum = jax.lax.dot_general(
        lmat_ref[...], zq, (((1,), (0,)), ((), ())),
        preferred_element_type=jnp.float32)
    colsum = cum[_D - 1 : _D, :]
    cfull = jax.lax.dot_general(
        colsum, fc_ref[...], (((1,), (1,)), ((), ())),
        preferred_element_type=jnp.float32)
    cumsel = jax.lax.dot_general(
        cum, qsel_ref[...], (((1,), (0,)), ((), ())),
        preferred_element_type=jnp.float32)
    corr = jnp.sum(wc_ref[...] * cumsel, axis=0, keepdims=True)
    cvec = cfull + corr
    seg = cvec[:, 1:] - cvec[:, :_D]
    part = jnp.sum(seg * seg)

    @pl.when(b == 0)
    def _init():
        loss_ref[0, 0] = part

    @pl.when(b != 0)
    def _acc():
        loss_ref[0, 0] += part


def kernel(z, codebook):
    z3 = z.reshape(_B, _D, _P)
    rvn = jnp.asarray(_rvn_bdp())
    lmat, fc, qsel, wc = (jnp.asarray(a) for a in _loss_consts())
    nb = _D + 1

    zq3, ind3, loss2 = pl.pallas_call(
        _nsvq_body,
        grid=(_B,),
        in_specs=[
            pl.BlockSpec((1, _D, _P), lambda b: (b, 0, 0)),
            pl.BlockSpec((_M, _D), lambda b: (0, 0)),
            pl.BlockSpec((1, _D, _P), lambda b: (b, 0, 0)),
            pl.BlockSpec((_D, _D), lambda b: (0, 0)),
            pl.BlockSpec((nb, _P), lambda b: (0, 0)),
            pl.BlockSpec((_P, nb), lambda b: (0, 0)),
            pl.BlockSpec((_D, nb), lambda b: (0, 0)),
        ],
        out_specs=[
            pl.BlockSpec((1, _D, _P), lambda b: (b, 0, 0)),
            pl.BlockSpec((1, 1, _P), lambda b: (b, 0, 0)),
            pl.BlockSpec(memory_space=pltpu.SMEM),
        ],
        out_shape=[
            jax.ShapeDtypeStruct((_B, _D, _P), jnp.float32),
            jax.ShapeDtypeStruct((_B, 1, _P), jnp.int32),
            jax.ShapeDtypeStruct((1, 1), jnp.float32),
        ],
    )(z3, codebook, rvn, lmat, fc, qsel, wc)

    z_q_out = zq3.reshape(_B, _D, _H, _W)
    ind = ind3.reshape(_N)
    loss = loss2[0, 0] / np.float32(_B * _P * _P)
    return (z_q_out, loss, ind)


# speedup vs baseline: 1.8298x; 1.0787x over previous
import numpy as np
import jax
import jax.numpy as jnp
from jax.experimental import pallas as pl
from jax.experimental.pallas import tpu as pltpu

_B, _D, _H, _W = 8, 64, 14, 14
_P = _H * _W
_N = _B * _P
_M = 1024

_consts = {}


def _threefry2x32_np(k1, k2, x0, x1):
    rot = [13, 15, 26, 6, 17, 29, 16, 24]
    ks = [np.uint32(k1), np.uint32(k2),
          np.uint32(k1) ^ np.uint32(k2) ^ np.uint32(0x1BD11BDA)]
    x0 = (x0 + ks[0]).astype(np.uint32)
    x1 = (x1 + ks[1]).astype(np.uint32)

    def rotl(v, d):
        return ((v << np.uint32(d)) | (v >> np.uint32(32 - d))).astype(np.uint32)

    for i in range(5):
        for j in range(4):
            x0 = (x0 + x1).astype(np.uint32)
            x1 = rotl(x1, rot[(i % 2) * 4 + j]) ^ x0
        x0 = (x0 + ks[(i + 1) % 3]).astype(np.uint32)
        x1 = (x1 + ks[(i + 2) % 3] + np.uint32(i + 1)).astype(np.uint32)
    return x0, x1


def _erfinv32_np(x):
    x = x.astype(np.float32)
    w = (-np.log1p((-x * x).astype(np.float32))).astype(np.float32)
    w1 = (w - np.float32(2.5)).astype(np.float32)
    p1 = np.float32(2.81022636e-08)
    for c in (3.43273939e-07, -3.5233877e-06, -4.39150654e-06, 0.00021858087,
              -0.00125372503, -0.00417768164, 0.246640727, 1.50140941):
        p1 = (np.float32(c) + p1 * w1).astype(np.float32)
    w2 = (np.sqrt(w.astype(np.float32)) - np.float32(3.0)).astype(np.float32)
    p2 = np.float32(-0.000200214257)
    for c in (0.000100950558, 0.00134934322, -0.00367342844, 0.00573950773,
              -0.0076224613, 0.00943887047, 1.00167406, 2.83297682):
        p2 = (np.float32(c) + p2 * w2).astype(np.float32)
    p = np.where(w < np.float32(5.0), p1, p2).astype(np.float32)
    return (p * x).astype(np.float32)


def _normal_threefry_np(seed, n):
    c0 = np.zeros(n, dtype=np.uint32)
    c1 = np.arange(n, dtype=np.uint32)
    x0, x1 = _threefry2x32_np(0, seed, c0, c1)
    bits = x0 ^ x1
    float_bits = (bits >> np.uint32(9)) | np.uint32(0x3F800000)
    floats = (float_bits.view(np.float32) - np.float32(1.0)).astype(np.float32)
    lo = np.nextafter(np.float32(-1.0), np.float32(0.0))
    hi = np.float32(1.0)
    u = np.maximum(lo, (floats * (hi - lo) + lo).astype(np.float32))
    return (np.float32(np.sqrt(2.0)) * _erfinv32_np(u)).astype(np.float32)


def _loss_consts():
    if "loss" not in _consts:
        nb = _D + 1
        q = [(_P * i) // _D for i in range(nb)]
        r = [(_P * i) % _D for i in range(nb)]
        lmat = np.tril(np.ones((_D, _D), np.float32))
        fc = np.zeros((nb, _P), np.float32)
        qsel = np.zeros((_P, nb), np.float32)
        wc = np.zeros((_D, nb), np.float32)
        for i in range(nb):
            fc[i, : q[i]] = 1.0
            if r[i] > 0:
                qsel[q[i], i] = 1.0
                wc[r[i] - 1, i] = 1.0
        _consts["loss"] = (lmat, fc, qsel, wc)
    return _consts["loss"]


def _rvn_bdp():
    if "rvn" not in _consts:
        rv = _normal_threefry_np(42, _N * _D).reshape(_N, _D)
        nrm = np.sqrt(np.sum(rv * rv, axis=1, keepdims=True, dtype=np.float32))
        nrm = np.maximum(nrm.astype(np.float32), np.float32(1e-9))
        rvn = (rv / nrm).astype(np.float32)
        _consts["rvn"] = np.ascontiguousarray(
            np.transpose(rvn.reshape(_B, _P, _D), (0, 2, 1))
        )
    return _consts["rvn"]


def _nsvq_body(z_ref, cb_ref, rvn_ref, rows_ref, lmat_ref, fc_ref, qsel_ref,
               wc_ref, zq_ref, ind_ref, loss_ref, y2_ref):
    b = pl.program_id(0)
    zb = z_ref[0]
    cb = cb_ref[...]

    @pl.when(b == 0)
    def _y2():
        y2_ref[...] = jnp.sum(cb * cb, axis=1, keepdims=True)

    g = jax.lax.dot_general(
        cb, zb, (((1,), (0,)), ((), ())), preferred_element_type=jnp.float32
    )
    x2 = jnp.sum(zb * zb, axis=0, keepdims=True)
    dist = (x2 + y2_ref[...]) - 2.0 * g

    dmin = jnp.min(dist, axis=0, keepdims=True)
    rows = jnp.broadcast_to(rows_ref[...], (_M, _P))
    indf = jnp.min(jnp.where(dist == dmin, rows, np.float32(_M)), axis=0)
    ind_ref[0, 0, :] = indf.astype(jnp.int32)

    s = jnp.sqrt(jnp.maximum(dmin, 0.0))
    zq = zb + rvn_ref[0] * s
    zq_ref[0] = zq

    cum = jax.lax.dot_general(
        lmat_ref[...], zq, (((1,), (0,)), ((), ())),
        preferred_element_type=jnp.float32)
    colsum = cum[_D - 1 : _D, :]
    cfull = jax.lax.dot_general(
        colsum, fc_ref[...], (((1,), (1,)), ((), ())),
        preferred_element_type=jnp.float32)
    cumsel = jax.lax.dot_general(
        cum, qsel_ref[...], (((1,), (0,)), ((), ())),
        preferred_element_type=jnp.float32)
    corr = jnp.sum(wc_ref[...] * cumsel, axis=0, keepdims=True)
    cvec = cfull + corr
    seg = cvec[:, 1:] - cvec[:, :_D]
    part = jnp.sum(seg * seg)

    @pl.when(b == 0)
    def _init():
        loss_ref[0, 0] = part

    @pl.when(jnp.logical_and(b != 0, b != _B - 1))
    def _acc():
        loss_ref[0, 0] += part

    @pl.when(b == _B - 1)
    def _fin():
        loss_ref[0, 0] = (loss_ref[0, 0] + part) / np.float32(_B * _P * _P)


def kernel(z, codebook):
    z3 = z.reshape(_B, _D, _P)
    rvn = jnp.asarray(_rvn_bdp())
    rows = jnp.asarray(np.arange(_M, dtype=np.float32).reshape(_M, 1))
    lmat, fc, qsel, wc = (jnp.asarray(a) for a in _loss_consts())
    nb = _D + 1

    zq3, ind3, loss2 = pl.pallas_call(
        _nsvq_body,
        grid=(_B,),
        in_specs=[
            pl.BlockSpec((1, _D, _P), lambda b: (b, 0, 0)),
            pl.BlockSpec((_M, _D), lambda b: (0, 0)),
            pl.BlockSpec((1, _D, _P), lambda b: (b, 0, 0)),
            pl.BlockSpec((_M, 1), lambda b: (0, 0)),
            pl.BlockSpec((_D, _D), lambda b: (0, 0)),
            pl.BlockSpec((nb, _P), lambda b: (0, 0)),
            pl.BlockSpec((_P, nb), lambda b: (0, 0)),
            pl.BlockSpec((_D, nb), lambda b: (0, 0)),
        ],
        scratch_shapes=[pltpu.VMEM((_M, 1), jnp.float32)],
        out_specs=[
            pl.BlockSpec((1, _D, _P), lambda b: (b, 0, 0)),
            pl.BlockSpec((1, 1, _P), lambda b: (b, 0, 0)),
            pl.BlockSpec(memory_space=pltpu.SMEM),
        ],
        out_shape=[
            jax.ShapeDtypeStruct((_B, _D, _P), jnp.float32),
            jax.ShapeDtypeStruct((_B, 1, _P), jnp.int32),
            jax.ShapeDtypeStruct((1, 1), jnp.float32),
        ],
    )(z3, codebook, rvn, rows, lmat, fc, qsel, wc)

    z_q_out = zq3.reshape(_B, _D, _H, _W)
    ind = ind3.reshape(_N)
    loss = loss2.reshape(())
    return (z_q_out, loss, ind)


# speedup vs baseline: 2.0773x; 1.1352x over previous
import numpy as np
import jax
import jax.numpy as jnp
from jax.experimental import pallas as pl
from jax.experimental.pallas import tpu as pltpu

_B, _D, _H, _W = 8, 64, 14, 14
_P = _H * _W
_N = _B * _P
_M = 1024

_consts = {}


def _threefry2x32_np(k1, k2, x0, x1):
    rot = [13, 15, 26, 6, 17, 29, 16, 24]
    ks = [np.uint32(k1), np.uint32(k2),
          np.uint32(k1) ^ np.uint32(k2) ^ np.uint32(0x1BD11BDA)]
    x0 = (x0 + ks[0]).astype(np.uint32)
    x1 = (x1 + ks[1]).astype(np.uint32)

    def rotl(v, d):
        return ((v << np.uint32(d)) | (v >> np.uint32(32 - d))).astype(np.uint32)

    for i in range(5):
        for j in range(4):
            x0 = (x0 + x1).astype(np.uint32)
            x1 = rotl(x1, rot[(i % 2) * 4 + j]) ^ x0
        x0 = (x0 + ks[(i + 1) % 3]).astype(np.uint32)
        x1 = (x1 + ks[(i + 2) % 3] + np.uint32(i + 1)).astype(np.uint32)
    return x0, x1


def _erfinv32_np(x):
    x = x.astype(np.float32)
    w = (-np.log1p((-x * x).astype(np.float32))).astype(np.float32)
    w1 = (w - np.float32(2.5)).astype(np.float32)
    p1 = np.float32(2.81022636e-08)
    for c in (3.43273939e-07, -3.5233877e-06, -4.39150654e-06, 0.00021858087,
              -0.00125372503, -0.00417768164, 0.246640727, 1.50140941):
        p1 = (np.float32(c) + p1 * w1).astype(np.float32)
    w2 = (np.sqrt(w.astype(np.float32)) - np.float32(3.0)).astype(np.float32)
    p2 = np.float32(-0.000200214257)
    for c in (0.000100950558, 0.00134934322, -0.00367342844, 0.00573950773,
              -0.0076224613, 0.00943887047, 1.00167406, 2.83297682):
        p2 = (np.float32(c) + p2 * w2).astype(np.float32)
    p = np.where(w < np.float32(5.0), p1, p2).astype(np.float32)
    return (p * x).astype(np.float32)


def _normal_threefry_np(seed, n):
    c0 = np.zeros(n, dtype=np.uint32)
    c1 = np.arange(n, dtype=np.uint32)
    x0, x1 = _threefry2x32_np(0, seed, c0, c1)
    bits = x0 ^ x1
    float_bits = (bits >> np.uint32(9)) | np.uint32(0x3F800000)
    floats = (float_bits.view(np.float32) - np.float32(1.0)).astype(np.float32)
    lo = np.nextafter(np.float32(-1.0), np.float32(0.0))
    hi = np.float32(1.0)
    u = np.maximum(lo, (floats * (hi - lo) + lo).astype(np.float32))
    return (np.float32(np.sqrt(2.0)) * _erfinv32_np(u)).astype(np.float32)


def _loss_consts():
    if "loss" not in _consts:
        nb = _D + 1
        q = [(_P * i) // _D for i in range(nb)]
        r = [(_P * i) % _D for i in range(nb)]
        fc = np.zeros((nb, _P), np.float32)
        qsel = np.zeros((_P, nb), np.float32)
        wd = np.zeros((_D, nb), np.float32)
        for i in range(nb):
            fc[i, : q[i]] = 1.0
            if r[i] > 0:
                qsel[q[i], i] = 1.0
                wd[: r[i], i] = 1.0
        _consts["loss"] = (fc, qsel, wd)
    return _consts["loss"]


def _rvn_bdp():
    if "rvn" not in _consts:
        rv = _normal_threefry_np(42, _N * _D).reshape(_N, _D)
        nrm = np.sqrt(np.sum(rv * rv, axis=1, keepdims=True, dtype=np.float32))
        nrm = np.maximum(nrm.astype(np.float32), np.float32(1e-9))
        rvn = (rv / nrm).astype(np.float32)
        _consts["rvn"] = np.ascontiguousarray(
            np.transpose(rvn.reshape(_B, _P, _D), (0, 2, 1))
        )
    return _consts["rvn"]


_TS = _B // 2 + 1


def _nsvq_body(zcur_ref, zlag_ref, cb_ref, rvn_ref, rows_ref, fc_ref, q_ref,
               wd_ref, zq_ref, ind_ref, loss_ref, g0_ref, g1_ref, y2_ref):
    t = pl.program_id(0)
    cb = cb_ref[...]

    @pl.when(t == 0)
    def _y2():
        y2_ref[...] = jnp.sum(cb * cb, axis=1, keepdims=True)

    parts = []
    for j, g_ref in enumerate((g0_ref, g1_ref)):
        zb = zlag_ref[j]
        g = g_ref[...]
        x2 = jnp.sum(zb * zb, axis=0, keepdims=True)
        dist = (x2 + y2_ref[...]) - 2.0 * g

        dmin = jnp.min(dist, axis=0, keepdims=True)
        rows = jnp.broadcast_to(rows_ref[...], (_M, _P))
        indf = jnp.min(jnp.where(dist == dmin, rows, np.float32(_M)), axis=0)
        ind_ref[j, 0, :] = indf.astype(jnp.int32)

        s = jnp.sqrt(jnp.maximum(dmin, 0.0))
        zq = zb + rvn_ref[j] * s
        zq_ref[j] = zq

        colsum = jnp.sum(zq, axis=0, keepdims=True)
        cfull = jax.lax.dot_general(
            colsum, fc_ref[...], (((1,), (1,)), ((), ())),
            preferred_element_type=jnp.float32)
        zsel = jax.lax.dot_general(
            zq, q_ref[...], (((1,), (0,)), ((), ())),
            preferred_element_type=jnp.float32)
        corr = jnp.sum(wd_ref[...] * zsel, axis=0, keepdims=True)
        cvec = cfull + corr
        seg = cvec[:, 1:] - cvec[:, :_D]
        parts.append(jnp.sum(seg * seg))

    part = parts[0] + parts[1]
    acc = jnp.where(t == 1, part, loss_ref[0, 0] + part)
    loss_ref[0, 0] = jnp.where(
        t == _TS - 1, acc / np.float32(_B * _P * _P), acc)

    g0_ref[...] = jax.lax.dot_general(
        cb, zcur_ref[0], (((1,), (0,)), ((), ())),
        preferred_element_type=jnp.float32)
    g1_ref[...] = jax.lax.dot_general(
        cb, zcur_ref[1], (((1,), (0,)), ((), ())),
        preferred_element_type=jnp.float32)


def kernel(z, codebook):
    z3 = z.reshape(_B, _D, _P)
    rvn = jnp.asarray(_rvn_bdp())
    rows = jnp.asarray(np.arange(_M, dtype=np.float32).reshape(_M, 1))
    fc, qsel, wd = (jnp.asarray(a) for a in _loss_consts())
    nb = _D + 1
    nblk = _B // 2 - 1

    zq3, ind3, loss2 = pl.pallas_call(
        _nsvq_body,
        grid=(_TS,),
        in_specs=[
            pl.BlockSpec((2, _D, _P), lambda t: (jnp.minimum(t, nblk), 0, 0)),
            pl.BlockSpec((2, _D, _P), lambda t: (jnp.maximum(t - 1, 0), 0, 0)),
            pl.BlockSpec((_M, _D), lambda t: (0, 0)),
            pl.BlockSpec((2, _D, _P), lambda t: (jnp.maximum(t - 1, 0), 0, 0)),
            pl.BlockSpec((_M, 1), lambda t: (0, 0)),
            pl.BlockSpec((nb, _P), lambda t: (0, 0)),
            pl.BlockSpec((_P, nb), lambda t: (0, 0)),
            pl.BlockSpec((_D, nb), lambda t: (0, 0)),
        ],
        scratch_shapes=[
            pltpu.VMEM((_M, _P), jnp.float32),
            pltpu.VMEM((_M, _P), jnp.float32),
            pltpu.VMEM((_M, 1), jnp.float32),
        ],
        out_specs=[
            pl.BlockSpec((2, _D, _P), lambda t: (jnp.maximum(t - 1, 0), 0, 0)),
            pl.BlockSpec((2, 1, _P), lambda t: (jnp.maximum(t - 1, 0), 0, 0)),
            pl.BlockSpec(memory_space=pltpu.SMEM),
        ],
        out_shape=[
            jax.ShapeDtypeStruct((_B, _D, _P), jnp.float32),
            jax.ShapeDtypeStruct((_B, 1, _P), jnp.int32),
            jax.ShapeDtypeStruct((1, 1), jnp.float32),
        ],
    )(z3, z3, codebook, rvn, rows, fc, qsel, wd)

    z_q_out = zq3.reshape(_B, _D, _H, _W)
    ind = ind3.reshape(_N)
    loss = loss2.reshape(())
    return (z_q_out, loss, ind)
